# Initial kernel scaffold; baseline (speedup 1.0000x reference)
import functools

import jax
import jax.numpy as jnp
import numpy as np
from jax.experimental import pallas as pl
from jax.experimental.pallas import tpu as pltpu

_CTX = 32
_C = 3
_HDIM = 64
_E = 16
_H = 384
_W = 384
_SYNTH = 6
_NPLANES = _C * _CTX + _SYNTH + _C
_CELL = 96
_NPIX_CELL = _CELL * _CELL


def _ctx_offsets():
    causal = np.arange(40)
    ys = causal // 9
    xs = causal % 9
    d = (ys - 4) ** 2 + (xs - 4) ** 2
    order = np.argsort(d, kind='stable')
    sel = np.sort(causal[order[:_CTX]])
    return sel // 9 - 4, sel % 9 - 4


def _plane_table():
    dy, dx = _ctx_offsets()
    chans, dys, dxs = [], [], []
    for c in range(_C):
        chans += [c] * _CTX
        dys += list(dy + 4)
        dxs += list(dx + 4)
    for s in range(_SYNTH):
        chans.append(_C + s)
        dys.append(4)
        dxs.append(4)
    for c in range(_C):
        chans.append(c)
        dys.append(4)
        dxs.append(4)
    return (np.asarray(chans, np.int32), np.asarray(dys, np.int32),
            np.asarray(dxs, np.int32))


def _im2col_kernel(offs_ref, src_ref, out_ref):
    k = pl.program_id(0)
    dy = offs_ref[k, 0]
    branches = [
        functools.partial(
            lambda i: src_ref[0, pl.ds(dy, _H), i:i + _W], i)
        for i in range(9)
    ]
    out_ref[0] = jax.lax.switch(offs_ref[k, 1], branches)


def _mlp_kernel(x_ref, w0_ref, b0_ref, wh_ref, bh_ref, wo_ref, bo_ref,
                out_ref):
    x = x_ref[0]
    for c in range(_C):
        h = jnp.dot(x, w0_ref[0, c], preferred_element_type=jnp.float32)
        h = jnp.maximum(h + b0_ref[0, c], 0.0)
        for l in range(2):
            hh = jnp.dot(h, wh_ref[0, c, l],
                         preferred_element_type=jnp.float32)
            h = jnp.maximum(h + hh + bh_ref[0, c, l], 0.0)
        y = jnp.dot(h, wo_ref[0, c], preferred_element_type=jnp.float32)
        out_ref[0, :, 4 * c:4 * (c + 1)] = y + bo_ref[0, c]


def kernel(image, raw_synth_out, W0_c0, b0_c0, Wh_c0, bh_c0, Wo_c0, bo_c0,
           W0_c1, b0_c1, Wh_c1, bh_c1, Wo_c1, bo_c1, W0_c2, b0_c2, Wh_c2,
           bh_c2, Wo_c2, bo_c2):
    chans, dys, dxs = _plane_table()
    offs = jnp.asarray(np.stack([dys, dxs], axis=1))

    src = jnp.pad(jnp.concatenate([image[0], raw_synth_out[0]], axis=0),
                  ((0, 0), (4, 4), (4, 4)))
    chan_map = jnp.asarray(chans)
    planes = pl.pallas_call(
        _im2col_kernel,
        grid=(_NPLANES,),
        in_specs=[
            pl.BlockSpec(memory_space=pltpu.SMEM),
            pl.BlockSpec((1, _H + 8, _W + 8),
                         lambda k: (chan_map[k], 0, 0)),
        ],
        out_specs=pl.BlockSpec((1, _H, _W), lambda k: (k, 0, 0)),
        out_shape=jax.ShapeDtypeStruct((_NPLANES, _H, _W), jnp.float32),
    )(offs, src)

    x_cells = (planes.reshape(_NPLANES, 4, _CELL, 4, _CELL)
               .transpose(1, 3, 2, 4, 0)
               .reshape(_E, _NPIX_CELL, _NPLANES))

    def pad_w0(w):
        return jnp.pad(w, ((0, 0), (0, _NPLANES - w.shape[1]), (0, 0)))

    w0 = jnp.stack([pad_w0(W0_c0), pad_w0(W0_c1), pad_w0(W0_c2)], 1)
    b0 = jnp.stack([b0_c0, b0_c1, b0_c2], 1)[:, :, None, :]
    wh = jnp.stack([Wh_c0, Wh_c1, Wh_c2], 1)
    bh = jnp.stack([bh_c0, bh_c1, bh_c2], 1)[:, :, :, None, :]
    wo = jnp.stack([Wo_c0, Wo_c1, Wo_c2], 1)
    bo = jnp.stack([bo_c0, bo_c1, bo_c2], 1)[:, :, None, :]

    out_cells = pl.pallas_call(
        _mlp_kernel,
        grid=(_E,),
        in_specs=[
            pl.BlockSpec((1, _NPIX_CELL, _NPLANES), lambda e: (e, 0, 0)),
            pl.BlockSpec((1, _C, _NPLANES, _HDIM), lambda e: (e, 0, 0, 0)),
            pl.BlockSpec((1, _C, 1, _HDIM), lambda e: (e, 0, 0, 0)),
            pl.BlockSpec((1, _C, 2, _HDIM, _HDIM),
                         lambda e: (e, 0, 0, 0, 0)),
            pl.BlockSpec((1, _C, 2, 1, _HDIM), lambda e: (e, 0, 0, 0, 0)),
            pl.BlockSpec((1, _C, _HDIM, 4), lambda e: (e, 0, 0, 0)),
            pl.BlockSpec((1, _C, 1, 4), lambda e: (e, 0, 0, 0)),
        ],
        out_specs=pl.BlockSpec((1, _NPIX_CELL, 4 * _C),
                               lambda e: (e, 0, 0)),
        out_shape=jax.ShapeDtypeStruct((_E, _NPIX_CELL, 4 * _C),
                                       jnp.float32),
    )(x_cells, w0, b0, wh, bh, wo, bo)

    return (out_cells.reshape(4, 4, _CELL, _CELL, _C, 4)
            .transpose(0, 2, 1, 3, 4, 5)
            .reshape(_H * _W, _C, 4))


# speedup vs baseline: 95.0172x; 95.0172x over previous
import functools

import jax
import jax.numpy as jnp
import numpy as np
from jax.experimental import pallas as pl
from jax.experimental.pallas import tpu as pltpu

_CTX = 32
_C = 3
_HDIM = 64
_E = 16
_H = 384
_W = 384
_SYNTH = 6
_NPLANES = _C * _CTX + _SYNTH + _C
_CELL = 96
_NPIX_CELL = _CELL * _CELL


def _ctx_offsets():
    causal = np.arange(40)
    ys = causal // 9
    xs = causal % 9
    d = (ys - 4) ** 2 + (xs - 4) ** 2
    order = np.argsort(d, kind='stable')
    sel = np.sort(causal[order[:_CTX]])
    return sel // 9 - 4, sel % 9 - 4


def _plane_table():
    dy, dx = _ctx_offsets()
    chans, dys, dxs = [], [], []
    for c in range(_C):
        chans += [c] * _CTX
        dys += list(dy + 4)
        dxs += list(dx + 4)
    for s in range(_SYNTH):
        chans.append(_C + s)
        dys.append(4)
        dxs.append(4)
    for c in range(_C):
        chans.append(c)
        dys.append(4)
        dxs.append(4)
    return (np.asarray(chans, np.int32), np.asarray(dys, np.int32),
            np.asarray(dxs, np.int32))


def _shift_pairs():
    chans, dys, dxs = _plane_table()
    pairs = sorted(set(zip(dys.tolist(), dxs.tolist())))
    lut = {p: i for i, p in enumerate(pairs)}
    sel = np.asarray([lut[(dy, dx)] for dy, dx in zip(dys, dxs)], np.int32)
    return pairs, sel


def _im2col_kernel(offs_ref, src_ref, out_ref):
    k = pl.program_id(0)
    pairs, _ = _shift_pairs()
    branches = [
        functools.partial(
            lambda y, x: src_ref[0, y:y + _H, x:x + _W], y, x)
        for (y, x) in pairs
    ]
    out_ref[0] = jax.lax.switch(offs_ref[k], branches)


def _mlp_kernel(x_ref, w0_ref, b0_ref, wh_ref, bh_ref, wo_ref, bo_ref,
                out_ref):
    x = x_ref[0]
    for c in range(_C):
        h = jnp.dot(x, w0_ref[0, c], preferred_element_type=jnp.float32)
        h = jnp.maximum(h + b0_ref[0, c], 0.0)
        for l in range(2):
            hh = jnp.dot(h, wh_ref[0, c, l],
                         preferred_element_type=jnp.float32)
            h = jnp.maximum(h + hh + bh_ref[0, c, l], 0.0)
        y = jnp.dot(h, wo_ref[0, c], preferred_element_type=jnp.float32)
        out_ref[0, :, 4 * c:4 * (c + 1)] = y + bo_ref[0, c]


def kernel(image, raw_synth_out, W0_c0, b0_c0, Wh_c0, bh_c0, Wo_c0, bo_c0,
           W0_c1, b0_c1, Wh_c1, bh_c1, Wo_c1, bo_c1, W0_c2, b0_c2, Wh_c2,
           bh_c2, Wo_c2, bo_c2):
    _, sel = _shift_pairs()
    offs = jnp.asarray(sel)

    src = jnp.pad(jnp.concatenate([image[0], raw_synth_out[0]], axis=0),
                  ((0, 0), (4, 4), (4, 4)))
    def src_chan(k):
        return jnp.where(k < 96, k // 32,
                         jnp.where(k < 102, k - 93, k - 102))

    planes = pl.pallas_call(
        _im2col_kernel,
        grid=(_NPLANES,),
        in_specs=[
            pl.BlockSpec(memory_space=pltpu.SMEM),
            pl.BlockSpec((1, _H + 8, _W + 8),
                         lambda k: (src_chan(k), 0, 0)),
        ],
        out_specs=pl.BlockSpec((1, _H, _W), lambda k: (k, 0, 0)),
        out_shape=jax.ShapeDtypeStruct((_NPLANES, _H, _W), jnp.float32),
    )(offs, src)

    x_cells = (planes.reshape(_NPLANES, 4, _CELL, 4, _CELL)
               .transpose(1, 3, 2, 4, 0)
               .reshape(_E, _NPIX_CELL, _NPLANES))

    def pad_w0(w):
        return jnp.pad(w, ((0, 0), (0, _NPLANES - w.shape[1]), (0, 0)))

    w0 = jnp.stack([pad_w0(W0_c0), pad_w0(W0_c1), pad_w0(W0_c2)], 1)
    b0 = jnp.stack([b0_c0, b0_c1, b0_c2], 1)[:, :, None, :]
    wh = jnp.stack([Wh_c0, Wh_c1, Wh_c2], 1)
    bh = jnp.stack([bh_c0, bh_c1, bh_c2], 1)[:, :, :, None, :]
    wo = jnp.stack([Wo_c0, Wo_c1, Wo_c2], 1)
    bo = jnp.stack([bo_c0, bo_c1, bo_c2], 1)[:, :, None, :]

    out_cells = pl.pallas_call(
        _mlp_kernel,
        grid=(_E,),
        in_specs=[
            pl.BlockSpec((1, _NPIX_CELL, _NPLANES), lambda e: (e, 0, 0)),
            pl.BlockSpec((1, _C, _NPLANES, _HDIM), lambda e: (e, 0, 0, 0)),
            pl.BlockSpec((1, _C, 1, _HDIM), lambda e: (e, 0, 0, 0)),
            pl.BlockSpec((1, _C, 2, _HDIM, _HDIM),
                         lambda e: (e, 0, 0, 0, 0)),
            pl.BlockSpec((1, _C, 2, 1, _HDIM), lambda e: (e, 0, 0, 0, 0)),
            pl.BlockSpec((1, _C, _HDIM, 4), lambda e: (e, 0, 0, 0)),
            pl.BlockSpec((1, _C, 1, 4), lambda e: (e, 0, 0, 0)),
        ],
        out_specs=pl.BlockSpec((1, _NPIX_CELL, 4 * _C),
                               lambda e: (e, 0, 0)),
        out_shape=jax.ShapeDtypeStruct((_E, _NPIX_CELL, 4 * _C),
                                       jnp.float32),
    )(x_cells, w0, b0, wh, bh, wo, bo)

    return (out_cells.reshape(4, 4, _CELL, _CELL, _C, 4)
            .transpose(0, 2, 1, 3, 4, 5)
            .reshape(_H * _W, _C, 4))


# speedup vs baseline: 148.9959x; 1.5681x over previous
import functools

import jax
import jax.numpy as jnp
import numpy as np
from jax.experimental import pallas as pl
from jax.experimental.pallas import tpu as pltpu

_CTX = 32
_C = 3
_HDIM = 64
_HD1 = _HDIM + 1
_E = 16
_H = 384
_W = 384
_SYNTH = 6
_NREAL = _C * _CTX + _SYNTH + _C
_NP = 112
_GROUP = 7
_NGROUPS = _NP // _GROUP
_CELL = 96
_NPIX_CELL = _CELL * _CELL


def _ctx_offsets():
    causal = np.arange(40)
    ys = causal // 9
    xs = causal % 9
    d = (ys - 4) ** 2 + (xs - 4) ** 2
    order = np.argsort(d, kind='stable')
    sel = np.sort(causal[order[:_CTX]])
    return sel // 9 - 4, sel % 9 - 4


def _plane_table():
    dy, dx = _ctx_offsets()
    table = []
    for c in range(_C):
        table += [(c, int(y) + 4, int(x) + 4) for y, x in zip(dy, dx)]
    table += [(_C + s, 4, 4) for s in range(_SYNTH)]
    table += [(c, 4, 4) for c in range(_C)]
    table += [(9, 4, 4)] * (_NP - _NREAL)
    return table


def _im2col_kernel(src_ref, out_ref):
    table = _plane_table()

    def copy_group(g):
        for i in range(_GROUP):
            c, y, x = table[_GROUP * g + i]
            out_ref[i] = src_ref[c, y:y + _H, x:x + _W]

    branches = [functools.partial(copy_group, g) for g in range(_NGROUPS)]
    jax.lax.switch(pl.program_id(0), branches)


def _mlp_kernel(x_ref, w0_ref, wh_ref, wo_ref, out_ref):
    x = x_ref[0]
    for c in range(_C):
        h = jnp.maximum(
            jnp.dot(x, w0_ref[0, c], preferred_element_type=jnp.float32),
            0.0)
        for l in range(2):
            h = jnp.maximum(
                jnp.dot(h, wh_ref[0, c, l],
                        preferred_element_type=jnp.float32), 0.0)
        out_ref[0, :, 4 * c:4 * (c + 1)] = jnp.dot(
            h, wo_ref[0, c], preferred_element_type=jnp.float32)


def kernel(image, raw_synth_out, W0_c0, b0_c0, Wh_c0, bh_c0, Wo_c0, bo_c0,
           W0_c1, b0_c1, Wh_c1, bh_c1, Wo_c1, bo_c1, W0_c2, b0_c2, Wh_c2,
           bh_c2, Wo_c2, bo_c2):
    f32 = jnp.float32

    src = jnp.pad(
        jnp.concatenate(
            [image[0], raw_synth_out[0],
             jnp.ones((1, _H, _W), f32)], axis=0),
        ((0, 0), (4, 4), (4, 4)))
    planes = pl.pallas_call(
        _im2col_kernel,
        grid=(_NGROUPS,),
        in_specs=[
            pl.BlockSpec((10, _H + 8, _W + 8), lambda g: (0, 0, 0)),
        ],
        out_specs=pl.BlockSpec((_GROUP, _H, _W), lambda g: (g, 0, 0)),
        out_shape=jax.ShapeDtypeStruct((_NP, _H, _W), f32),
    )(src)

    x_cells = (planes.reshape(_NP, 4, _CELL, 4, _CELL)
               .transpose(1, 3, 2, 4, 0)
               .reshape(_E, _NPIX_CELL, _NP))

    def pack_w0(w, b):
        w = jnp.pad(w, ((0, 0), (0, _NREAL - w.shape[1]), (0, 0)))
        w = jnp.concatenate(
            [w, b[:, None, :], jnp.zeros((_E, _NP - _NREAL - 1, _HDIM),
                                         f32)], axis=1)
        ones_col = np.zeros((_NP, 1), np.float32)
        ones_col[_NREAL] = 1.0
        return jnp.concatenate(
            [w, jnp.broadcast_to(jnp.asarray(ones_col), (_E, _NP, 1))],
            axis=2)

    def pack_wh(w, b):
        w = w + jnp.eye(_HDIM, dtype=f32)
        top = jnp.concatenate(
            [w, jnp.zeros((_E, 2, _HDIM, 1), f32)], axis=3)
        bot = jnp.concatenate(
            [b, jnp.ones((_E, 2, 1), f32)], axis=2)[:, :, None, :]
        return jnp.concatenate([top, bot], axis=2)

    def pack_wo(w, b):
        return jnp.concatenate([w, b[:, None, :]], axis=1)

    w0 = jnp.stack([pack_w0(W0_c0, b0_c0), pack_w0(W0_c1, b0_c1),
                    pack_w0(W0_c2, b0_c2)], 1)
    wh = jnp.stack([pack_wh(Wh_c0, bh_c0), pack_wh(Wh_c1, bh_c1),
                    pack_wh(Wh_c2, bh_c2)], 1)
    wo = jnp.stack([pack_wo(Wo_c0, bo_c0), pack_wo(Wo_c1, bo_c1),
                    pack_wo(Wo_c2, bo_c2)], 1)

    out_cells = pl.pallas_call(
        _mlp_kernel,
        grid=(_E,),
        in_specs=[
            pl.BlockSpec((1, _NPIX_CELL, _NP), lambda e: (e, 0, 0)),
            pl.BlockSpec((1, _C, _NP, _HD1), lambda e: (e, 0, 0, 0)),
            pl.BlockSpec((1, _C, 2, _HD1, _HD1),
                         lambda e: (e, 0, 0, 0, 0)),
            pl.BlockSpec((1, _C, _HD1, 4), lambda e: (e, 0, 0, 0)),
        ],
        out_specs=pl.BlockSpec((1, _NPIX_CELL, 4 * _C),
                               lambda e: (e, 0, 0)),
        out_shape=jax.ShapeDtypeStruct((_E, _NPIX_CELL, 4 * _C), f32),
    )(x_cells, w0, wh, wo)

    return (out_cells.reshape(4, 4, _CELL, _CELL, _C, 4)
            .transpose(0, 2, 1, 3, 4, 5)
            .reshape(_H * _W, _C, 4))
